# Initial kernel scaffold; baseline (speedup 1.0000x reference)
import jax
import jax.numpy as jnp
from jax.experimental import pallas as pl


def kernel(H1, H2, H3, edge_index, params):
    raise NotImplementedError("write your pallas kernel here")


# speedup vs baseline: 3.0157x; 3.0157x over previous
import jax
import jax.numpy as jnp
from jax.experimental import pallas as pl
from jax.experimental.pallas import tpu as pltpu
from math import sqrt

N = 10000
D = 128
_B = 1000


def _final_body(h1, h2, h3, linw, linb, d2w1, d2b1, d2w2, d2b2,
                d3w1, d3b1, d3w2, d3b2, hcat, out2, out3):
    a = h1[...]
    b = h2[...]
    c = h3[...]
    hcat[:, 0:D] = a
    hcat[:, D:2 * D] = b
    hcat[:, 2 * D:3 * D] = c
    h = jnp.maximum(
        a @ linw[0:D, :] + b @ linw[D:2 * D, :] + c @ linw[2 * D:3 * D, :]
        + linb[...], 0.0)
    o2 = jnp.maximum(h @ d2w1[...] + d2b1[...], 0.0)
    o2 = jnp.maximum(o2 @ d2w2[...] + d2b2[...], 0.0)
    out2[...] = o2
    o3 = jnp.maximum(h @ d3w1[...] + d3b1[...], 0.0)
    o3 = jnp.maximum(o3 @ d3w2[...] + d3b2[...], 0.0)
    out3[...] = o3


def _final_stage(H1, H2, H3, params):
    linw, linb = params["lin"]
    (d2w1, d2b1), (d2w2, d2b2) = params["dec2"]
    (d3w1, d3b1), (d3w2, d3b2) = params["dec3"]
    row = lambda i: (i, 0)
    full = lambda i: (0, 0)
    wspec = pl.BlockSpec((D, D), full)
    bspec = pl.BlockSpec((1, D), full)
    return pl.pallas_call(
        _final_body,
        grid=(N // _B,),
        in_specs=[
            pl.BlockSpec((_B, D), row),
            pl.BlockSpec((_B, D), row),
            pl.BlockSpec((_B, D), row),
            pl.BlockSpec((3 * D, D), full),
            bspec,
            wspec, bspec, wspec, bspec,
            wspec, bspec, wspec, bspec,
        ],
        out_specs=[
            pl.BlockSpec((_B, 3 * D), row),
            pl.BlockSpec((_B, D), row),
            pl.BlockSpec((_B, D), row),
        ],
        out_shape=[
            jax.ShapeDtypeStruct((N, 3 * D), jnp.float32),
            jax.ShapeDtypeStruct((N, D), jnp.float32),
            jax.ShapeDtypeStruct((N, D), jnp.float32),
        ],
    )(H1, H2, H3, linw, linb.reshape(1, D),
      d2w1, d2b1.reshape(1, D), d2w2, d2b2.reshape(1, D),
      d3w1, d3b1.reshape(1, D), d3w2, d3b2.reshape(1, D))


def _segment_softmax_agg(q, k, src, dst, num_nodes):
    qe = jnp.take(q, dst, axis=0)
    ke = jnp.take(k, src, axis=0)
    score = jnp.sum(qe * ke, axis=1) / sqrt(q.shape[1])
    ex = jnp.exp(score)
    denom = jax.ops.segment_sum(ex, dst, num_segments=num_nodes)
    acc = jax.ops.segment_sum(ex[:, None] * ke, dst, num_segments=num_nodes)
    return acc / (denom + 1e-16)[:, None]


def _cross_attention(lp, src, dst, H1, H2, H3):
    n = H1.shape[0]
    Z1 = H1 @ lp["l1"][0] + lp["l1"][1]
    Z2 = H2 @ lp["l2"][0] + lp["l2"][1]
    Z3 = H3 @ lp["l3"][0] + lp["l3"][1]
    O1 = jax.nn.relu(_segment_softmax_agg(Z1, 0.5 * (Z2 + Z3), src, dst, n))
    O2 = jax.nn.relu(_segment_softmax_agg(Z2, 0.5 * (Z1 + Z3), src, dst, n))
    O3 = jax.nn.relu(_segment_softmax_agg(Z3, 0.5 * (Z1 + Z2), src, dst, n))
    return O1, O2, O3


def kernel(H1, H2, H3, edge_index, params):
    src, dst = edge_index[0], edge_index[1]
    H1 = H1 @ params["t1"][0] + params["t1"][1]
    H2 = H2 @ params["t2"][0] + params["t2"][1]
    H3 = H3 @ params["t3"][0] + params["t3"][1]
    for lp in params["enc"]:
        H1, H2, H3 = _cross_attention(lp, src, dst, H1, H2, H3)
    return _final_stage(H1, H2, H3, params)


# speedup vs baseline: 8.9330x; 2.9621x over previous
import functools
from math import sqrt

import jax
import jax.numpy as jnp
from jax import lax
from jax.experimental import pallas as pl
from jax.experimental.pallas import tpu as pltpu
from jax.experimental.pallas import tpu_sc as plsc

N = 10000
D = 128
NP = 10240
STRIPE = NP // 16
E = 320000
CH = 80
NCHUNK = 128
IG = 8
EP = 32 * NCHUNK * CH
_INV = 1.0 / sqrt(128.0)


def _prep1_body(h1, h2, h3, t1w, t1b, t2w, t2b, t3w, t3b,
                l1w, l1b, l2w, l2b, l3w, l3b,
                q1, q2, q3, k1, k2, k3):
    a1 = h1[...] @ t1w[...] + t1b[...]
    a2 = h2[...] @ t2w[...] + t2b[...]
    a3 = h3[...] @ t3w[...] + t3b[...]
    z1 = a1 @ l1w[...] + l1b[...]
    z2 = a2 @ l2w[...] + l2b[...]
    z3 = a3 @ l3w[...] + l3b[...]
    q1[...] = z1
    q2[...] = z2
    q3[...] = z3
    k1[...] = 0.5 * (z2 + z3)
    k2[...] = 0.5 * (z1 + z3)
    k3[...] = 0.5 * (z1 + z2)


def _prep1(Hp1, Hp2, Hp3, params):
    row = lambda i: (i, 0)
    full = lambda i: (0, 0)
    wspec = pl.BlockSpec((D, D), full)
    bspec = pl.BlockSpec((1, D), full)
    hspec = pl.BlockSpec((STRIPE, D), row)
    ospec = pl.BlockSpec((STRIPE, D), row)
    oshape = jax.ShapeDtypeStruct((NP, D), jnp.float32)
    lp = params["enc"][0]
    args = [Hp1, Hp2, Hp3]
    for nm in ("t1", "t2", "t3"):
        w, b = params[nm]
        args += [w, b.reshape(1, D)]
    for nm in ("l1", "l2", "l3"):
        w, b = lp[nm]
        args += [w, b.reshape(1, D)]
    return pl.pallas_call(
        _prep1_body,
        grid=(16,),
        in_specs=[hspec] * 3 + [wspec, bspec] * 6,
        out_specs=[ospec] * 6,
        out_shape=[oshape] * 6,
    )(*args)


def _edge_body(q1, q2, q3, k1, k2, k3, srcr, dstr, zrows, zvec, out, dout,
               sbuf, dbuf, qbuf, kbuf, wkbuf, wvals, accsh, densh,
               semq, semk, sema, semd):
    cid = lax.axis_index("c")
    sid = lax.axis_index("s")
    zero16 = jnp.zeros((16,), jnp.float32)
    iot = lax.iota(jnp.int32, 16)
    onehot = [jnp.where(iot == i, 1.0, 0.0).astype(jnp.float32)
              for i in range(16)]

    for a, (Q, K) in enumerate(((q1, k1), (q2, k2), (q3, k3))):
        pltpu.sync_copy(zrows, accsh.at[pl.ds(sid * STRIPE, STRIPE)])
        pltpu.sync_copy(zvec, densh.at[pl.ds(sid * STRIPE, STRIPE)])
        plsc.subcore_barrier()

        def slab_body(g, carry, Q=Q, K=K):
            pltpu.sync_copy(srcr.at[cid, sid, pl.ds(g * IG, IG)], sbuf)
            pltpu.sync_copy(dstr.at[cid, sid, pl.ds(g * IG, IG)], dbuf)

            def chunk_body(j, c0, Q=Q, K=K):
                si = sbuf.at[j]
                di = dbuf.at[j]
                cq = pltpu.async_copy(Q.at[di], qbuf, semq)
                ck = pltpu.async_copy(K.at[si], kbuf, semk)

                @pl.when(g * IG + j > 0)
                def _():
                    pltpu.make_async_copy(wkbuf, accsh.at[di], sema).wait()
                    pltpu.make_async_copy(wvals, densh.at[di], semd).wait()

                cq.wait()
                ck.wait()

                @plsc.parallel_loop(0, CH // 16, unroll=1)
                def grp(g2):
                    base = g2 * 16
                    wv = zero16
                    for i in range(16):
                        e = base + i
                        pd = qbuf[e, pl.ds(0, 16)] * kbuf[e, pl.ds(0, 16)]
                        for jj in range(1, 8):
                            pd = pd + qbuf[e, pl.ds(16 * jj, 16)] * kbuf[e, pl.ds(16 * jj, 16)]
                        s = jnp.sum(pd) * _INV
                        w = jnp.exp(jnp.broadcast_to(s, (16,)))
                        for jj in range(8):
                            wkbuf[e, pl.ds(16 * jj, 16)] = w * kbuf[e, pl.ds(16 * jj, 16)]
                        wv = wv + w * onehot[i]
                    wvals[pl.ds(base, 16)] = wv

                pltpu.async_copy(wkbuf, accsh.at[di], sema, add=True)
                pltpu.async_copy(wvals, densh.at[di], semd, add=True)
                return c0

            lax.fori_loop(0, IG, chunk_body, 0)
            return carry

        lax.fori_loop(0, NCHUNK // IG, slab_body, 0)
        pltpu.make_async_copy(wkbuf, accsh.at[dbuf.at[0]], sema).wait()
        pltpu.make_async_copy(wvals, densh.at[dbuf.at[0]], semd).wait()
        plsc.subcore_barrier()
        pltpu.sync_copy(accsh.at[pl.ds(sid * STRIPE, STRIPE)],
                        out.at[a, cid, pl.ds(sid * STRIPE, STRIPE)])
        pltpu.sync_copy(densh.at[pl.ds(sid * STRIPE, STRIPE)],
                        dout.at[a, cid, pl.ds(sid * STRIPE, STRIPE)])
        plsc.subcore_barrier()


def _edge_sc(q1, q2, q3, k1, k2, k3, srcr, dstr, zrows, zvec):
    mesh = plsc.VectorSubcoreMesh(core_axis_name="c", subcore_axis_name="s")
    fn = functools.partial(
        pl.kernel,
        out_type=(jax.ShapeDtypeStruct((3, 2, NP, D), jnp.float32),
                  jax.ShapeDtypeStruct((3, 2, NP), jnp.float32)),
        mesh=mesh,
        compiler_params=pltpu.CompilerParams(needs_layout_passes=False),
        scratch_types=[
            pltpu.VMEM((IG, CH), jnp.int32),
            pltpu.VMEM((IG, CH), jnp.int32),
            pltpu.VMEM((CH, D), jnp.float32),
            pltpu.VMEM((CH, D), jnp.float32),
            pltpu.VMEM((CH, D), jnp.float32),
            pltpu.VMEM((CH,), jnp.float32),
            pltpu.VMEM_SHARED((NP, D), jnp.float32),
            pltpu.VMEM_SHARED((NP,), jnp.float32),
            pltpu.SemaphoreType.DMA,
            pltpu.SemaphoreType.DMA,
            pltpu.SemaphoreType.DMA,
            pltpu.SemaphoreType.DMA,
        ],
    )(_edge_body)
    return fn(q1, q2, q3, k1, k2, k3, srcr, dstr, zrows, zvec)


def _norm3(accv, denv):
    os = []
    for i in range(3):
        num = accv[i, 0] + accv[i, 1]
        den = denv[i, 0] + denv[i, 1]
        os.append(jnp.maximum(num / (den[:, None] + 1e-16), 0.0))
    return os


def _merge_body(acc, den, l1w, l1b, l2w, l2b, l3w, l3b, q1, q2, q3, k1, k2, k3):
    o1, o2, o3 = _norm3(acc[...], den[...])
    z1 = o1 @ l1w[...] + l1b[...]
    z2 = o2 @ l2w[...] + l2b[...]
    z3 = o3 @ l3w[...] + l3b[...]
    q1[...] = z1
    q2[...] = z2
    q3[...] = z3
    k1[...] = 0.5 * (z2 + z3)
    k2[...] = 0.5 * (z1 + z3)
    k3[...] = 0.5 * (z1 + z2)


def _merge_prep(acc, den, lp):
    full = lambda i: (0, 0)
    wspec = pl.BlockSpec((D, D), full)
    bspec = pl.BlockSpec((1, D), full)
    aspec = pl.BlockSpec((3, 2, STRIPE, D), lambda i: (0, 0, i, 0))
    dspec = pl.BlockSpec((3, 2, STRIPE), lambda i: (0, 0, i))
    ospec = pl.BlockSpec((STRIPE, D), lambda i: (i, 0))
    oshape = jax.ShapeDtypeStruct((NP, D), jnp.float32)
    args = [acc, den]
    for nm in ("l1", "l2", "l3"):
        w, b = lp[nm]
        args += [w, b.reshape(1, D)]
    return pl.pallas_call(
        _merge_body,
        grid=(16,),
        in_specs=[aspec, dspec] + [wspec, bspec] * 3,
        out_specs=[ospec] * 6,
        out_shape=[oshape] * 6,
    )(*args)


def _final_body(acc, den, linw, linb, d2w1, d2b1, d2w2, d2b2,
                d3w1, d3b1, d3w2, d3b2, hcat, out2, out3):
    o1, o2, o3 = _norm3(acc[...], den[...])
    hcat[:, 0:D] = o1
    hcat[:, D:2 * D] = o2
    hcat[:, 2 * D:3 * D] = o3
    h = jnp.maximum(
        o1 @ linw[0:D, :] + o2 @ linw[D:2 * D, :] + o3 @ linw[2 * D:3 * D, :]
        + linb[...], 0.0)
    a2 = jnp.maximum(h @ d2w1[...] + d2b1[...], 0.0)
    out2[...] = jnp.maximum(a2 @ d2w2[...] + d2b2[...], 0.0)
    a3 = jnp.maximum(h @ d3w1[...] + d3b1[...], 0.0)
    out3[...] = jnp.maximum(a3 @ d3w2[...] + d3b2[...], 0.0)


def _final(acc, den, params):
    B = 1024
    full = lambda i: (0, 0)
    wspec = pl.BlockSpec((D, D), full)
    bspec = pl.BlockSpec((1, D), full)
    aspec = pl.BlockSpec((3, 2, B, D), lambda i: (0, 0, i, 0))
    dspec = pl.BlockSpec((3, 2, B), lambda i: (0, 0, i))
    linw, linb = params["lin"]
    (d2w1, d2b1), (d2w2, d2b2) = params["dec2"]
    (d3w1, d3b1), (d3w2, d3b2) = params["dec3"]
    return pl.pallas_call(
        _final_body,
        grid=(pl.cdiv(N, B),),
        in_specs=[aspec, dspec, pl.BlockSpec((3 * D, D), full), bspec,
                  wspec, bspec, wspec, bspec, wspec, bspec, wspec, bspec],
        out_specs=[
            pl.BlockSpec((B, 3 * D), lambda i: (i, 0)),
            pl.BlockSpec((B, D), lambda i: (i, 0)),
            pl.BlockSpec((B, D), lambda i: (i, 0)),
        ],
        out_shape=[
            jax.ShapeDtypeStruct((N, 3 * D), jnp.float32),
            jax.ShapeDtypeStruct((N, D), jnp.float32),
            jax.ShapeDtypeStruct((N, D), jnp.float32),
        ],
    )(acc, den, linw, linb.reshape(1, D),
      d2w1, d2b1.reshape(1, D), d2w2, d2b2.reshape(1, D),
      d3w1, d3b1.reshape(1, D), d3w2, d3b2.reshape(1, D))


def kernel(H1, H2, H3, edge_index, params):
    pad_rows = jnp.zeros((NP - N, D), jnp.float32)
    Hp1 = jnp.concatenate([H1, pad_rows])
    Hp2 = jnp.concatenate([H2, pad_rows])
    Hp3 = jnp.concatenate([H3, pad_rows])
    src = edge_index[0].astype(jnp.int32)
    dst = edge_index[1].astype(jnp.int32)
    pad_e = jnp.full((EP - E,), N, jnp.int32)
    srcr = jnp.concatenate([src, pad_e]).reshape(2, 16, NCHUNK, CH)
    dstr = jnp.concatenate([dst, pad_e]).reshape(2, 16, NCHUNK, CH)
    del pad_rows
    zrows = jnp.zeros((STRIPE, D), jnp.float32)
    zvec = jnp.zeros((STRIPE,), jnp.float32)

    q1, q2, q3, k1, k2, k3 = _prep1(Hp1, Hp2, Hp3, params)
    acc, den = _edge_sc(q1, q2, q3, k1, k2, k3, srcr, dstr, zrows, zvec)
    q1, q2, q3, k1, k2, k3 = _merge_prep(acc, den, params["enc"][1])
    acc, den = _edge_sc(q1, q2, q3, k1, k2, k3, srcr, dstr, zrows, zvec)
    return _final(acc, den, params)
